# baseline (speedup 1.0000x reference)
import functools

import jax
import jax.numpy as jnp
from jax import lax
from jax.experimental import pallas as pl
from jax.experimental.pallas import tpu as pltpu
from jax.experimental.pallas import tpu_sc as plsc

B, M, K, DSUB = 16384, 96, 256, 8
D = M * DSUB
HALVES = 2
MH = M // HALVES
DH = D // HALVES
TW = MH * K * DSUB
MW = K * DSUB
NSUB = 16
ROWS_PER_SUB = B // NSUB
CHUNK = 16
NCHUNK = ROWS_PER_SUB // CHUNK
NACC = 5
JV = DH // 16


def _sc_body(q_hbm, pos_hbm, neg_hbm, tab_hbm, out_hbm,
             tab_v, qb, pb, nb, ob, sem0, sem1):
    c = lax.axis_index("c")
    s = lax.axis_index("s")
    base_row = s * ROWS_PER_SUB

    pltpu.sync_copy(tab_hbm.at[c], tab_v)

    iota = lax.broadcasted_iota(jnp.int32, (16,), 0)
    hi = lax.shift_right_logical(iota, 3)
    d8 = lax.bitwise_and(iota, 7)
    tpat = hi * MW + d8 * K

    def fire(g, slot, sem, colq0):
        r0 = base_row + g * CHUNK
        dst = pl.ds(slot * CHUNK, CHUNK)
        pltpu.async_copy(q_hbm.at[pl.ds(r0, CHUNK), pl.ds(colq0, DH)],
                         qb.at[dst], sem)
        pltpu.async_copy(pos_hbm.at[pl.ds(r0, CHUNK)], pb.at[dst], sem)
        pltpu.async_copy(neg_hbm.at[pl.ds(r0, CHUNK)], nb.at[dst], sem)

    def wait(g, slot, sem, colq0):
        r0 = base_row + g * CHUNK
        dst = pl.ds(slot * CHUNK, CHUNK)
        pltpu.make_async_copy(
            q_hbm.at[pl.ds(r0, CHUNK), pl.ds(colq0, DH)],
            qb.at[dst], sem).wait()
        pltpu.make_async_copy(pos_hbm.at[pl.ds(r0, CHUNK)],
                              pb.at[dst], sem).wait()
        pltpu.make_async_copy(neg_hbm.at[pl.ds(r0, CHUNK)],
                              nb.at[dst], sem).wait()

    def compute(g, slot, colc0):
        def row_body(r, carry):
            srow = slot * CHUNK + r
            rsplat = jnp.full((16,), srow, jnp.int32)
            q2 = jnp.zeros((16,), jnp.float32)
            dpp = jnp.zeros((16,), jnp.float32)
            dpn = jnp.zeros((16,), jnp.float32)
            n2p = jnp.zeros((16,), jnp.float32)
            n2n = jnp.zeros((16,), jnp.float32)
            for j in range(JV):
                qv = qb[srow, pl.ds(16 * j, 16)]
                q2 = q2 + qv * qv
                cidx = (colc0 + 2 * j) + hi
                cp = plsc.load_gather(pb, [rsplat, cidx])
                cn = plsc.load_gather(nb, [rsplat, cidx])
                toff = tpat + (2 * j) * MW
                tp = plsc.load_gather(tab_v, [cp + toff])
                tn = plsc.load_gather(tab_v, [cn + toff])
                dpp = dpp + qv * tp
                n2p = n2p + tp * tp
                dpn = dpn + qv * tn
                n2n = n2n + tn * tn
            rglob = g * CHUNK + r
            lane0 = iota == 0
            sums = (jnp.sum(q2), jnp.sum(dpp), jnp.sum(dpn),
                    jnp.sum(n2p), jnp.sum(n2n))
            for i in range(NACC):
                idx = jnp.full((16,), i * ROWS_PER_SUB + rglob, jnp.int32)
                plsc.store_scatter(ob, [idx],
                                   jnp.full((16,), sums[i], jnp.float32),
                                   mask=lane0)
            return carry
        lax.fori_loop(0, CHUNK, row_body, 0)

    def half(colq0, colc0):
        fire(0, 0, sem0, colq0)
        fire(1, 1, sem1, colq0)

        def outer(g2, carry):
            g = 2 * g2
            wait(g, 0, sem0, colq0)
            compute(g, 0, colc0)

            @pl.when(g2 < NCHUNK // 2 - 1)
            def _():
                fire(g + 2, 0, sem0, colq0)

            wait(g + 1, 1, sem1, colq0)
            compute(g + 1, 1, colc0)

            @pl.when(g2 < NCHUNK // 2 - 1)
            def _():
                fire(g + 3, 1, sem1, colq0)

            return carry

        lax.fori_loop(0, NCHUNK // 2, outer, 0)

    @pl.when(c == 0)
    def _():
        half(0, 0)

    @pl.when(c == 1)
    def _():
        half(DH, MH)

    pltpu.sync_copy(ob, out_hbm.at[c, s])


_sc_partials = functools.partial(
    pl.kernel,
    out_type=jax.ShapeDtypeStruct((HALVES, NSUB, NACC * ROWS_PER_SUB),
                                  jnp.float32),
    mesh=plsc.VectorSubcoreMesh(core_axis_name="c", subcore_axis_name="s"),
    compiler_params=pltpu.CompilerParams(
        use_tc_tiling_on_sc=True, needs_layout_passes=False),
    scratch_types=[
        pltpu.VMEM((TW,), jnp.float32),
        pltpu.VMEM((2 * CHUNK, DH), jnp.float32),
        pltpu.VMEM((2 * CHUNK, M), jnp.int32),
        pltpu.VMEM((2 * CHUNK, M), jnp.int32),
        pltpu.VMEM((NACC * ROWS_PER_SUB,), jnp.float32),
        pltpu.SemaphoreType.DMA,
        pltpu.SemaphoreType.DMA,
    ],
)(_sc_body)


def _loss_body(x_ref, o_ref):
    x = x_ref[...]
    y = x[0] + x[1]
    R = ROWS_PER_SUB
    q2 = y[:, 0 * R:1 * R]
    dpp = y[:, 1 * R:2 * R]
    dpn = y[:, 2 * R:3 * R]
    n2p = y[:, 3 * R:4 * R]
    n2n = y[:, 4 * R:5 * R]
    eps = 1e-8
    nq = jnp.maximum(jnp.sqrt(q2), eps)
    sp = dpp / (nq * jnp.maximum(jnp.sqrt(n2p), eps))
    sn = dpn / (nq * jnp.maximum(jnp.sqrt(n2n), eps))
    mx = jnp.maximum(sp, sn)
    lse = jnp.log(jnp.exp(sp - mx) + jnp.exp(sn - mx)) + mx
    o_ref[0, 0] = jnp.sum(lse - sp) * (1.0 / B)


def kernel(q, pos_codes, neg_codes, codebooks):
    tab = codebooks.transpose(0, 2, 1).reshape(HALVES, TW)
    x = _sc_partials(q, pos_codes, neg_codes, tab)
    loss = pl.pallas_call(
        _loss_body,
        out_specs=pl.BlockSpec(memory_space=pltpu.SMEM),
        out_shape=jax.ShapeDtypeStruct((1, 1), jnp.float32),
    )(x)
    return loss[0, 0]
